# baseline (speedup 1.0000x reference)
import functools

import jax
import jax.numpy as jnp
from jax import lax
from jax.experimental import pallas as pl
from jax.experimental.pallas import tpu as pltpu
from jax.experimental.pallas import tpu_sc as plsc


def _make_pool_kernel(B, C, V, D):
    info = plsc.get_sparse_core_info()
    NC, NS, L = info.num_cores, info.num_subcores, info.num_lanes
    NW = NC * NS
    assert B % NW == 0 and D % L == 0
    b_per_w = B // NW
    n_idx = b_per_w * C
    IDX_CHUNK = 128
    assert n_idx % IDX_CHUNK == 0
    n_chunks = n_idx // IDX_CHUNK
    mesh = plsc.VectorSubcoreMesh(core_axis_name="c", subcore_axis_name="s")
    inv_c = 1.0 / C

    @functools.partial(
        pl.kernel,
        mesh=mesh,
        compiler_params=pltpu.CompilerParams(use_tc_tiling_on_sc=False),
        out_type=jax.ShapeDtypeStruct((B, D), jnp.float32),
        scratch_types=[
            pltpu.VMEM((n_idx,), jnp.int32),
            pltpu.VMEM((n_idx, D), jnp.float32),
            pltpu.VMEM((b_per_w, D), jnp.float32),
            pltpu.SemaphoreType.DMA,
        ],
    )
    def pool(ctx_hbm, table_hbm, out_hbm, idx_v, rows_v, acc_v, sem):
        wid = lax.axis_index("s") * NC + lax.axis_index("c")
        pltpu.sync_copy(ctx_hbm.at[pl.ds(wid * n_idx, n_idx)], idx_v)
        copies = [
            pltpu.async_copy(
                table_hbm.at[idx_v.at[pl.ds(k * IDX_CHUNK, IDX_CHUNK)]],
                rows_v.at[pl.ds(k * IDX_CHUNK, IDX_CHUNK)],
                sem,
            )
            for k in range(n_chunks)
        ]
        for cp in copies:
            cp.wait()

        def body(b, carry):
            for d in range(D // L):
                acc = jnp.zeros((L,), jnp.float32)
                for c in range(C):
                    acc = acc + rows_v[b * C + c, pl.ds(d * L, L)]
                acc_v[b, pl.ds(d * L, L)] = acc * inv_c
            return carry

        lax.fori_loop(0, b_per_w, body, 0)
        pltpu.sync_copy(acc_v, out_hbm.at[pl.ds(wid * b_per_w, b_per_w)])

    return pool


def _matmul(pooled, W, BV=2000):
    B, D = pooled.shape
    V = W.shape[0]

    def mm(w_ref, a_ref, o_ref):
        o_ref[...] = lax.dot_general(
            w_ref[...], a_ref[...],
            (((1,), (1,)), ((), ())),
            preferred_element_type=jnp.float32,
        )

    out_t = pl.pallas_call(
        mm,
        grid=(pl.cdiv(V, BV),),
        in_specs=[
            pl.BlockSpec((BV, D), lambda i: (i, 0)),
            pl.BlockSpec((B, D), lambda i: (0, 0)),
        ],
        out_specs=pl.BlockSpec((BV, B), lambda i: (i, 0)),
        out_shape=jax.ShapeDtypeStruct((V, B), jnp.float32),
        compiler_params=pltpu.CompilerParams(
            dimension_semantics=("parallel",),
        ),
    )(W, pooled)
    return out_t.T


@jax.jit
def kernel(context, emb_table, W):
    B, C = context.shape
    V, D = emb_table.shape
    ctx_flat = context.reshape(-1).astype(jnp.int32)
    pooled = _make_pool_kernel(B, C, V, D)(ctx_flat, emb_table)
    return _matmul(pooled, W)


# speedup vs baseline: 1.0327x; 1.0327x over previous
import functools

import jax
import jax.numpy as jnp
from jax import lax
from jax.experimental import pallas as pl
from jax.experimental.pallas import tpu as pltpu
from jax.experimental.pallas import tpu_sc as plsc


def _make_pool_kernel(B, C, V, D):
    info = plsc.get_sparse_core_info()
    NC, NS, L = info.num_cores, info.num_subcores, info.num_lanes
    NW = NC * NS
    assert B % NW == 0 and D % L == 0
    b_per_w = B // NW
    n_idx = b_per_w * C
    IDX_CHUNK = 128
    assert n_idx % IDX_CHUNK == 0
    n_chunks = n_idx // IDX_CHUNK
    mesh = plsc.VectorSubcoreMesh(core_axis_name="c", subcore_axis_name="s")
    inv_c = 1.0 / C

    @functools.partial(
        pl.kernel,
        mesh=mesh,
        compiler_params=pltpu.CompilerParams(use_tc_tiling_on_sc=False),
        out_type=jax.ShapeDtypeStruct((B, D), jnp.float32),
        scratch_types=[
            pltpu.VMEM((n_idx,), jnp.int32),
            pltpu.VMEM((n_idx, D), jnp.float32),
            pltpu.VMEM((b_per_w, D), jnp.float32),
            pltpu.SemaphoreType.DMA,
        ],
    )
    def pool(ctx_hbm, table_hbm, out_hbm, idx_v, rows_v, acc_v, sem):
        wid = lax.axis_index("s") * NC + lax.axis_index("c")
        pltpu.sync_copy(ctx_hbm.at[pl.ds(wid * n_idx, n_idx)], idx_v)
        copies = [
            pltpu.async_copy(
                table_hbm.at[idx_v.at[pl.ds(k * IDX_CHUNK, IDX_CHUNK)]],
                rows_v.at[pl.ds(k * IDX_CHUNK, IDX_CHUNK)],
                sem,
            )
            for k in range(n_chunks)
        ]
        for cp in copies:
            cp.wait()

        def body(b, carry):
            for d in range(D // L):
                acc = jnp.zeros((L,), jnp.float32)
                for c in range(C):
                    acc = acc + rows_v[b * C + c, pl.ds(d * L, L)]
                acc_v[b, pl.ds(d * L, L)] = acc * inv_c
            return carry

        lax.fori_loop(0, b_per_w, body, 0)
        pltpu.sync_copy(acc_v, out_hbm.at[pl.ds(wid * b_per_w, b_per_w)])

    return pool


def _matmul(pooled, W, BV=2000):
    B, D = pooled.shape
    V = W.shape[0]

    def mm(w_ref, a_ref, o_ref):
        o_ref[...] = lax.dot_general(
            w_ref[...], a_ref[...],
            (((1,), (1,)), ((), ())),
            preferred_element_type=jnp.float32,
        )

    out_t = pl.pallas_call(
        mm,
        grid=(pl.cdiv(V, BV),),
        in_specs=[
            pl.BlockSpec((BV, D), lambda i: (i, 0)),
            pl.BlockSpec((B, D), lambda i: (0, 0)),
        ],
        out_specs=pl.BlockSpec((BV, B), lambda i: (i, 0)),
        out_shape=jax.ShapeDtypeStruct((V, B), jnp.float32),
        compiler_params=pltpu.CompilerParams(
            dimension_semantics=("parallel",),
        ),
    )(W, pooled)
    return out_t.T


@jax.jit
def kernel(context, emb_table, W):
    B, C = context.shape
    V, D = emb_table.shape
    ctx_flat = context.reshape(-1).astype(jnp.int32)
    pooled = _make_pool_kernel(B, C, V, D)(ctx_flat, emb_table)
    return _matmul(pooled.astype(jnp.bfloat16), W.astype(jnp.bfloat16))
